# baseline (speedup 1.0000x reference)
import jax
import jax.numpy as jnp
from jax.experimental import pallas as pl

N = 10000
ALPHA = 0.2
EPS = 1e-5


def _lrelu_kernel(hn_ref, o_ref):
    hn = hn_ref[...]
    o_ref[...] = jnp.where(hn >= 0, hn, ALPHA * hn)


def kernel(x, edge_idx, edge_attr, W, b, gamma, beta):
    h = x @ W
    src = edge_idx[0]
    dst = edge_idx[1]
    loop = jnp.arange(N, dtype=src.dtype)
    src_f = jnp.concatenate([src, loop])
    dst_f = jnp.concatenate([dst, loop])
    w_f = jnp.concatenate([edge_attr, jnp.ones((N,), dtype=edge_attr.dtype)])
    deg = jax.ops.segment_sum(w_f, dst_f, num_segments=N)
    deg_inv_sqrt = jnp.where(deg > 0, 1.0 / jnp.sqrt(deg), 0.0)
    norm = deg_inv_sqrt[src_f] * w_f * deg_inv_sqrt[dst_f]
    msgs = h[src_f] * norm[:, None]
    agg = jax.ops.segment_sum(msgs, dst_f, num_segments=N) + b
    mean = jnp.mean(agg, axis=0)
    var = jnp.var(agg, axis=0)
    hn = (agg - mean) / jnp.sqrt(var + EPS) * gamma + beta
    return pl.pallas_call(
        _lrelu_kernel,
        out_shape=jax.ShapeDtypeStruct(hn.shape, hn.dtype),
        grid=(10,),
        in_specs=[pl.BlockSpec((1000, 128), lambda i: (i, 0))],
        out_specs=pl.BlockSpec((1000, 128), lambda i: (i, 0)),
    )(hn)


# speedup vs baseline: 23.5379x; 23.5379x over previous
import functools

import jax
import jax.numpy as jnp
from jax import lax
from jax.experimental import pallas as pl
from jax.experimental.pallas import tpu as pltpu
from jax.experimental.pallas import tpu_sc as plsc

N = 10000
N_PAD = 10240
E = 320000
D = 128
ALPHA = 0.2
EPS = 1e-5

NW = 32
C = 128
NCH = 80
E_PAD = NW * NCH * C
RPT = N_PAD // 16

_sc_mesh = plsc.VectorSubcoreMesh(core_axis_name="c", subcore_axis_name="s")


@functools.partial(
    pl.kernel,
    out_type=jax.ShapeDtypeStruct((2, 16, RPT), jnp.float32),
    mesh=_sc_mesh,
    scratch_types=[
        pltpu.VMEM((NCH, C), jnp.int32),
        pltpu.VMEM((NCH, C), jnp.float32),
        pltpu.VMEM_SHARED((N_PAD,), jnp.float32),
    ],
)
def _deg_kernel(dst_hbm, attr_hbm, zdeg_hbm, degp_hbm, dstbuf, attrbuf, deg_sh):
    c = lax.axis_index("c")
    s = lax.axis_index("s")
    wid = s * 2 + c
    pltpu.sync_copy(zdeg_hbm, deg_sh.at[pl.ds(s * RPT, RPT)])
    plsc.subcore_barrier()
    pltpu.sync_copy(dst_hbm.at[wid], dstbuf)
    pltpu.sync_copy(attr_hbm.at[wid], attrbuf)

    def chunk(j, carry):
        pltpu.sync_copy(attrbuf.at[j], deg_sh.at[dstbuf.at[j]], add=True)
        return carry

    lax.fori_loop(0, NCH, chunk, 0)
    plsc.subcore_barrier()
    pltpu.sync_copy(deg_sh.at[pl.ds(s * RPT, RPT)], degp_hbm.at[c, s])


def _lin_body(x_ref, w_ref, degp_ref, g_ref, dis_ref):
    h = jnp.dot(x_ref[...], w_ref[...], preferred_element_type=jnp.float32)
    deg = degp_ref[0] + degp_ref[1] + 1.0
    dis = lax.rsqrt(deg)
    g_ref[...] = h * dis
    dis_ref[...] = dis


def _lin_call(x_pad, W, degp):
    return pl.pallas_call(
        _lin_body,
        grid=(16,),
        in_specs=[
            pl.BlockSpec((640, D), lambda i: (i, 0)),
            pl.BlockSpec((D, D), lambda i: (0, 0)),
            pl.BlockSpec((2, 640, 1), lambda i: (0, i, 0)),
        ],
        out_specs=[
            pl.BlockSpec((640, D), lambda i: (i, 0)),
            pl.BlockSpec((640, 1), lambda i: (i, 0)),
        ],
        out_shape=[
            jax.ShapeDtypeStruct((N_PAD, D), jnp.float32),
            jax.ShapeDtypeStruct((N_PAD, 1), jnp.float32),
        ],
    )(x_pad, W, degp)


@functools.partial(
    pl.kernel,
    out_type=jax.ShapeDtypeStruct((2, 16, RPT, D), jnp.float32),
    mesh=_sc_mesh,
    scratch_types=[
        pltpu.VMEM((NCH, C), jnp.int32),
        pltpu.VMEM((NCH, C), jnp.int32),
        pltpu.VMEM((NCH * C,), jnp.float32),
        pltpu.VMEM((C, D), jnp.float32),
        pltpu.VMEM_SHARED((N_PAD, D), jnp.float32),
        pltpu.SemaphoreType.DMA,
    ],
)
def _scat_kernel(g_hbm, src_hbm, dst_hbm, attr_hbm, zrows_hbm, sp_hbm,
                 srcbuf, dstbuf, attrbuf, rows, s_sh, sem):
    c = lax.axis_index("c")
    s = lax.axis_index("s")
    wid = s * 2 + c
    pltpu.sync_copy(zrows_hbm, s_sh.at[pl.ds(s * RPT, RPT)])
    plsc.subcore_barrier()
    pltpu.sync_copy(src_hbm.at[wid], srcbuf)
    pltpu.sync_copy(dst_hbm.at[wid], dstbuf)
    pltpu.sync_copy(attr_hbm.at[wid], attrbuf)

    def chunk(j, carry):
        pltpu.async_copy(g_hbm.at[srcbuf.at[j]], rows, sem).wait()

        def scale_grp(grp, carry2):
            base = grp * 16
            avs = attrbuf[pl.ds(j * C + base, 16)]
            for dr in range(16):
                av = jnp.full((16,), avs[dr], jnp.float32)
                for f in range(8):
                    sl = pl.ds(f * 16, 16)
                    rows[base + dr, sl] = rows[base + dr, sl] * av
            return carry2

        lax.fori_loop(0, C // 16, scale_grp, 0)
        pltpu.sync_copy(rows, s_sh.at[dstbuf.at[j]], add=True)
        return carry

    lax.fori_loop(0, NCH, chunk, 0)
    plsc.subcore_barrier()
    pltpu.sync_copy(s_sh.at[pl.ds(s * RPT, RPT)], sp_hbm.at[c, s])


def _agg_body(g_ref, sp_ref, dis_ref, b_ref, agg_ref, sum_ref, sumsq_ref):
    i = pl.program_id(0)
    sblk = sp_ref[0] + sp_ref[1]
    agg = dis_ref[...] * (g_ref[...] + sblk) + b_ref[...]
    agg_ref[...] = agg
    s0 = jnp.sum(agg, axis=0, keepdims=True)
    s1 = jnp.sum(agg * agg, axis=0, keepdims=True)

    @pl.when(i == 0)
    def _():
        sum_ref[...] = s0
        sumsq_ref[...] = s1

    @pl.when(i > 0)
    def _():
        sum_ref[...] += s0
        sumsq_ref[...] += s1


def _agg_call(g, sp, dis, b2):
    return pl.pallas_call(
        _agg_body,
        grid=(10,),
        in_specs=[
            pl.BlockSpec((1000, D), lambda i: (i, 0)),
            pl.BlockSpec((2, 1000, D), lambda i: (0, i, 0)),
            pl.BlockSpec((1000, 1), lambda i: (i, 0)),
            pl.BlockSpec((1, D), lambda i: (0, 0)),
        ],
        out_specs=[
            pl.BlockSpec((1000, D), lambda i: (i, 0)),
            pl.BlockSpec((1, D), lambda i: (0, 0)),
            pl.BlockSpec((1, D), lambda i: (0, 0)),
        ],
        out_shape=[
            jax.ShapeDtypeStruct((N, D), jnp.float32),
            jax.ShapeDtypeStruct((1, D), jnp.float32),
            jax.ShapeDtypeStruct((1, D), jnp.float32),
        ],
    )(g, sp, dis, b2)


def _bn_body(agg_ref, sum_ref, sumsq_ref, gamma_ref, beta_ref, o_ref):
    mean = sum_ref[...] * (1.0 / N)
    var = sumsq_ref[...] * (1.0 / N) - mean * mean
    inv = lax.rsqrt(var + EPS)
    hn = (agg_ref[...] - mean) * inv * gamma_ref[...] + beta_ref[...]
    o_ref[...] = jnp.where(hn >= 0, hn, ALPHA * hn)


def _bn_call(agg, s0, s1, gamma2, beta2):
    return pl.pallas_call(
        _bn_body,
        grid=(10,),
        in_specs=[
            pl.BlockSpec((1000, D), lambda i: (i, 0)),
            pl.BlockSpec((1, D), lambda i: (0, 0)),
            pl.BlockSpec((1, D), lambda i: (0, 0)),
            pl.BlockSpec((1, D), lambda i: (0, 0)),
            pl.BlockSpec((1, D), lambda i: (0, 0)),
        ],
        out_specs=pl.BlockSpec((1000, D), lambda i: (i, 0)),
        out_shape=jax.ShapeDtypeStruct((N, D), jnp.float32),
    )(agg, s0, s1, gamma2, beta2)


def kernel(x, edge_idx, edge_attr, W, b, gamma, beta):
    src = edge_idx[0]
    dst = edge_idx[1]
    pad = E_PAD - E
    ar = jnp.arange(pad, dtype=jnp.int32)
    src_p = jnp.concatenate([src, ar % N]).reshape(NW, NCH, C)
    dst_p = jnp.concatenate([dst, N + ar % (N_PAD - N)]).reshape(NW, NCH, C)
    attr_p = jnp.concatenate(
        [edge_attr, jnp.zeros((pad,), jnp.float32)]).reshape(NW, NCH, C)
    zdeg = jnp.zeros((RPT,), jnp.float32)
    zrows = jnp.zeros((RPT, D), jnp.float32)
    x_pad = jnp.pad(x, ((0, N_PAD - N), (0, 0)))

    degp = _deg_kernel(dst_p, attr_p, zdeg).reshape(2, N_PAD, 1)
    g, dis = _lin_call(x_pad, W, degp)
    sp = _scat_kernel(g, src_p, dst_p, attr_p.reshape(NW, NCH * C),
                      zrows).reshape(2, N_PAD, D)
    agg, s0, s1 = _agg_call(g, sp, dis, b.reshape(1, D))
    return _bn_call(agg, s0, s1, gamma.reshape(1, D), beta.reshape(1, D))


# speedup vs baseline: 29.4816x; 1.2525x over previous
import functools

import jax
import jax.numpy as jnp
from jax import lax
from jax.experimental import pallas as pl
from jax.experimental.pallas import tpu as pltpu
from jax.experimental.pallas import tpu_sc as plsc

N = 10000
N_PAD = 10240
E = 320000
D = 128
ALPHA = 0.2
EPS = 1e-5

NW = 32
C = 128
NCH = 80
E_PAD = NW * NCH * C
RPT = N_PAD // 16

_sc_mesh = plsc.VectorSubcoreMesh(core_axis_name="c", subcore_axis_name="s")


@functools.partial(
    pl.kernel,
    out_type=jax.ShapeDtypeStruct((2, 16, RPT), jnp.float32),
    mesh=_sc_mesh,
    scratch_types=[
        pltpu.VMEM((NCH, C), jnp.int32),
        pltpu.VMEM((NCH, C), jnp.float32),
        pltpu.VMEM_SHARED((N_PAD,), jnp.float32),
    ],
)
def _deg_kernel(dst_hbm, attr_hbm, zdeg_hbm, degp_hbm, dstbuf, attrbuf, deg_sh):
    c = lax.axis_index("c")
    s = lax.axis_index("s")
    wid = s * 2 + c
    pltpu.sync_copy(zdeg_hbm, deg_sh.at[pl.ds(s * RPT, RPT)])
    plsc.subcore_barrier()
    pltpu.sync_copy(dst_hbm.at[wid], dstbuf)
    pltpu.sync_copy(attr_hbm.at[wid], attrbuf)

    def chunk(j, carry):
        pltpu.sync_copy(attrbuf.at[j], deg_sh.at[dstbuf.at[j]], add=True)
        return carry

    lax.fori_loop(0, NCH, chunk, 0)
    plsc.subcore_barrier()
    pltpu.sync_copy(deg_sh.at[pl.ds(s * RPT, RPT)], degp_hbm.at[c, s])


def _lin_body(x_ref, w_ref, degp_ref, g_ref, dis_ref):
    h = jnp.dot(x_ref[...], w_ref[...], preferred_element_type=jnp.float32)
    deg = degp_ref[0] + degp_ref[1] + 1.0
    dis = lax.rsqrt(deg)
    g_ref[...] = h * dis
    dis_ref[...] = dis


def _lin_call(x_pad, W, degp):
    return pl.pallas_call(
        _lin_body,
        grid=(16,),
        in_specs=[
            pl.BlockSpec((640, D), lambda i: (i, 0)),
            pl.BlockSpec((D, D), lambda i: (0, 0)),
            pl.BlockSpec((2, 640, 1), lambda i: (0, i, 0)),
        ],
        out_specs=[
            pl.BlockSpec((640, D), lambda i: (i, 0)),
            pl.BlockSpec((640, 1), lambda i: (i, 0)),
        ],
        out_shape=[
            jax.ShapeDtypeStruct((N_PAD, D), jnp.float32),
            jax.ShapeDtypeStruct((N_PAD, 1), jnp.float32),
        ],
    )(x_pad, W, degp)


@functools.partial(
    pl.kernel,
    out_type=jax.ShapeDtypeStruct((2, 16, RPT, D), jnp.float32),
    mesh=_sc_mesh,
    scratch_types=[
        pltpu.VMEM((2, 2, C), jnp.int32),
        pltpu.VMEM((NCH * C,), jnp.float32),
        pltpu.VMEM((C, D), jnp.float32),
        pltpu.VMEM((C, D), jnp.float32),
        pltpu.VMEM_SHARED((N_PAD, D), jnp.float32),
        pltpu.SemaphoreType.DMA,
        pltpu.SemaphoreType.DMA,
        pltpu.SemaphoreType.DMA,
        pltpu.SemaphoreType.DMA,
        pltpu.SemaphoreType.DMA,
    ],
)
def _scat_kernel(g_hbm, sd_hbm, attr_hbm, zrows_hbm, sp_hbm,
                 idxbuf, attrbuf, rows0, rows1, s_sh,
                 sem_i, sem_g0, sem_g1, sem_s0, sem_s1):
    c = lax.axis_index("c")
    s = lax.axis_index("s")
    wid = s * 2 + c
    pltpu.sync_copy(zrows_hbm, s_sh.at[pl.ds(s * RPT, RPT)])
    plsc.subcore_barrier()
    pltpu.sync_copy(attr_hbm.at[wid], attrbuf)

    rows = (rows0, rows1)
    sem_g = (sem_g0, sem_g1)
    sem_s = (sem_s0, sem_s1)

    def scale(j, buf):
        def scale_grp(grp, carry2):
            base = grp * 16
            avs = attrbuf[pl.ds(j * C + base, 16)]
            for dr in range(16):
                av = jnp.full((16,), avs[dr], jnp.float32)
                for f in range(8):
                    sl = pl.ds(f * 16, 16)
                    buf[base + dr, sl] = buf[base + dr, sl] * av
            return carry2

        lax.fori_loop(0, C // 16, scale_grp, 0)

    for b in (0, 1):
        pltpu.async_copy(sd_hbm.at[wid, b], idxbuf.at[b], sem_i).wait()
        pltpu.async_copy(g_hbm.at[idxbuf.at[b, 0]], rows[b], sem_g[b])

    def pair(jj, carry):
        j0 = jj * 2
        j1 = j0 + 1
        for b, j in ((0, j0), (1, j1)):
            pltpu.make_async_copy(g_hbm.at[idxbuf.at[b, 0]], rows[b], sem_g[b]).wait()
            scale(j, rows[b])
            pltpu.async_copy(rows[b], s_sh.at[idxbuf.at[b, 1]], sem_s[b], add=True)
        for b, j in ((0, j0), (1, j1)):
            pltpu.make_async_copy(rows[b], s_sh.at[idxbuf.at[b, 1]], sem_s[b]).wait()

            @pl.when(jj < NCH // 2 - 1)
            def _():
                pltpu.async_copy(sd_hbm.at[wid, j + 2], idxbuf.at[b], sem_i).wait()
                pltpu.async_copy(g_hbm.at[idxbuf.at[b, 0]], rows[b], sem_g[b])

        return carry

    lax.fori_loop(0, NCH // 2, pair, 0)
    plsc.subcore_barrier()
    pltpu.sync_copy(s_sh.at[pl.ds(s * RPT, RPT)], sp_hbm.at[c, s])


def _agg_body(g_ref, sp_ref, dis_ref, b_ref, agg_ref, sum_ref, sumsq_ref):
    i = pl.program_id(0)
    sblk = sp_ref[0] + sp_ref[1]
    agg = dis_ref[...] * (g_ref[...] + sblk) + b_ref[...]
    agg_ref[...] = agg
    s0 = jnp.sum(agg, axis=0, keepdims=True)
    s1 = jnp.sum(agg * agg, axis=0, keepdims=True)

    @pl.when(i == 0)
    def _():
        sum_ref[...] = s0
        sumsq_ref[...] = s1

    @pl.when(i > 0)
    def _():
        sum_ref[...] += s0
        sumsq_ref[...] += s1


def _agg_call(g, sp, dis, b2):
    return pl.pallas_call(
        _agg_body,
        grid=(10,),
        in_specs=[
            pl.BlockSpec((1000, D), lambda i: (i, 0)),
            pl.BlockSpec((2, 1000, D), lambda i: (0, i, 0)),
            pl.BlockSpec((1000, 1), lambda i: (i, 0)),
            pl.BlockSpec((1, D), lambda i: (0, 0)),
        ],
        out_specs=[
            pl.BlockSpec((1000, D), lambda i: (i, 0)),
            pl.BlockSpec((1, D), lambda i: (0, 0)),
            pl.BlockSpec((1, D), lambda i: (0, 0)),
        ],
        out_shape=[
            jax.ShapeDtypeStruct((N, D), jnp.float32),
            jax.ShapeDtypeStruct((1, D), jnp.float32),
            jax.ShapeDtypeStruct((1, D), jnp.float32),
        ],
    )(g, sp, dis, b2)


def _bn_body(agg_ref, sum_ref, sumsq_ref, gamma_ref, beta_ref, o_ref):
    mean = sum_ref[...] * (1.0 / N)
    var = sumsq_ref[...] * (1.0 / N) - mean * mean
    inv = lax.rsqrt(var + EPS)
    hn = (agg_ref[...] - mean) * inv * gamma_ref[...] + beta_ref[...]
    o_ref[...] = jnp.where(hn >= 0, hn, ALPHA * hn)


def _bn_call(agg, s0, s1, gamma2, beta2):
    return pl.pallas_call(
        _bn_body,
        grid=(10,),
        in_specs=[
            pl.BlockSpec((1000, D), lambda i: (i, 0)),
            pl.BlockSpec((1, D), lambda i: (0, 0)),
            pl.BlockSpec((1, D), lambda i: (0, 0)),
            pl.BlockSpec((1, D), lambda i: (0, 0)),
            pl.BlockSpec((1, D), lambda i: (0, 0)),
        ],
        out_specs=pl.BlockSpec((1000, D), lambda i: (i, 0)),
        out_shape=jax.ShapeDtypeStruct((N, D), jnp.float32),
    )(agg, s0, s1, gamma2, beta2)


def kernel(x, edge_idx, edge_attr, W, b, gamma, beta):
    src = edge_idx[0]
    dst = edge_idx[1]
    pad = E_PAD - E
    ar = jnp.arange(pad, dtype=jnp.int32)
    src_p = jnp.concatenate([src, ar % N]).reshape(NW, NCH, C)
    dst_p = jnp.concatenate([dst, N + ar % (N_PAD - N)]).reshape(NW, NCH, C)
    attr_p = jnp.concatenate(
        [edge_attr, jnp.zeros((pad,), jnp.float32)]).reshape(NW, NCH, C)
    zdeg = jnp.zeros((RPT,), jnp.float32)
    zrows = jnp.zeros((RPT, D), jnp.float32)
    x_pad = jnp.pad(x, ((0, N_PAD - N), (0, 0)))

    degp = _deg_kernel(dst_p, attr_p, zdeg).reshape(2, N_PAD, 1)
    g, dis = _lin_call(x_pad, W, degp)
    sd_p = jnp.stack([src_p, dst_p], axis=2)
    sp = _scat_kernel(g, sd_p, attr_p.reshape(NW, NCH * C),
                      zrows).reshape(2, N_PAD, D)
    agg, s0, s1 = _agg_call(g, sp, dis, b.reshape(1, D))
    return _bn_call(agg, s0, s1, gamma.reshape(1, D), beta.reshape(1, D))
